# Initial kernel scaffold; baseline (speedup 1.0000x reference)
import jax
import jax.numpy as jnp
from jax.experimental import pallas as pl


def kernel(token_ids, embedding_matrix):
    raise NotImplementedError("write your pallas kernel here")


# speedup vs baseline: 1.8763x; 1.8763x over previous
import functools

import jax
import jax.numpy as jnp
from jax import lax
from jax.experimental import pallas as pl
from jax.experimental.pallas import tpu as pltpu
from jax.experimental.pallas import tpu_sc as plsc

D = 64
CHUNK = 128
NBUF = 4


@functools.lru_cache(maxsize=None)
def _build(B: int):
    info = plsc.get_sparse_core_info()
    NW = info.num_cores * info.num_subcores
    b_per_w = B // NW
    n_chunks = b_per_w // CHUNK
    n_groups = n_chunks // NBUF
    mesh = plsc.VectorSubcoreMesh(core_axis_name="c", subcore_axis_name="s")

    @functools.partial(
        pl.kernel,
        mesh=mesh,
        out_type=jax.ShapeDtypeStruct((B, D), jnp.float32),
        compiler_params=pltpu.CompilerParams(use_tc_tiling_on_sc=False),
        scratch_types=(
            [pltpu.VMEM((n_chunks, CHUNK), jnp.int32),
             pltpu.VMEM((NBUF, CHUNK, D), jnp.float32)]
            + [pltpu.SemaphoreType.DMA] * (2 * NBUF)
        ),
    )
    def gather_kernel(idx_hbm, table_hbm, out_hbm, idx_v, rows_v, *sems):
        sem_g = sems[:NBUF]
        sem_w = sems[NBUF:]
        wid = lax.axis_index("s") * info.num_cores + lax.axis_index("c")
        base = wid * b_per_w

        pltpu.sync_copy(idx_hbm.at[wid], idx_v)

        def issue_gather(j, b):
            pltpu.async_copy(table_hbm.at[idx_v.at[j]], rows_v.at[b], sem_g[b])

        def wait_gather(j, b):
            pltpu.make_async_copy(
                table_hbm.at[idx_v.at[j]], rows_v.at[b], sem_g[b]).wait()

        def issue_write(j, b):
            pltpu.async_copy(
                rows_v.at[b], out_hbm.at[pl.ds(base + j * CHUNK, CHUNK)],
                sem_w[b])

        def wait_write(j, b):
            pltpu.make_async_copy(
                rows_v.at[b], out_hbm.at[pl.ds(base + j * CHUNK, CHUNK)],
                sem_w[b]).wait()

        for b in range(NBUF):
            issue_gather(b, b)

        def body(i, carry):
            for b in range(NBUF):
                j = i * NBUF + b
                wait_gather(j, b)
                issue_write(j, b)
                wait_write(j, b)
                issue_gather(j + NBUF, b)
            return carry

        lax.fori_loop(0, n_groups - 1, body, 0)

        for b in range(NBUF):
            j = (n_groups - 1) * NBUF + b
            wait_gather(j, b)
            issue_write(j, b)
        for b in range(NBUF):
            j = (n_groups - 1) * NBUF + b
            wait_write(j, b)

    return gather_kernel, NW, n_chunks


def kernel(token_ids, embedding_matrix):
    lead_shape = token_ids.shape
    flat = token_ids.reshape(-1).astype(jnp.int32)
    B = flat.shape[0]
    gather_kernel, NW, n_chunks = _build(B)
    idx3 = flat.reshape(NW, n_chunks, CHUNK)
    out = gather_kernel(idx3, embedding_matrix)
    return out.reshape(*lead_shape, D)
